# baseline (speedup 1.0000x reference)
import functools

import jax
import jax.numpy as jnp
from jax import lax
from jax.experimental import pallas as pl
from jax.experimental.pallas import tpu as pltpu
from jax.experimental.pallas import tpu_sc as plsc

_NC, _NS = 1, 16
_NW = _NC * _NS
_B = 2
_HW = 256
_Q = _B * _HW
_D = 256
_N = 512
_QPW = _Q // _NW


def _tc_scores_body(x_ref, w_ref, amin_ref, wt_ref):
    w = w_ref[...]
    e2 = jnp.sum(w * w, axis=0, keepdims=True)
    for b in range(_B):
        xb = x_ref[b]
        s = lax.dot_general(
            xb, w, (((0,), (0,)), ((), ())),
            preferred_element_type=jnp.float32,
            precision=lax.Precision.HIGHEST)
        score = e2 - 2.0 * s
        mn = jnp.min(score, axis=1, keepdims=True)
        ii = lax.broadcasted_iota(jnp.int32, score.shape, 1)
        amin_ref[b, :] = jnp.min(jnp.where(score <= mn, ii, _N), axis=1)
    wt_ref[...] = w.T


_tc_scores = pl.pallas_call(
    _tc_scores_body,
    out_shape=(
        jax.ShapeDtypeStruct((_B, _HW), jnp.int32),
        jax.ShapeDtypeStruct((_N, _D), jnp.float32),
    ),
)


def _sc_gather_body(wt_hbm, idx_hbm, out_hbm, idx_v, rows_v, sem):
    wid = lax.axis_index("s") * _NC + lax.axis_index("c")
    base = wid * _QPW
    pltpu.sync_copy(idx_hbm.at[pl.ds(base, _QPW)], idx_v)
    pltpu.async_copy(wt_hbm.at[idx_v], rows_v, sem).wait()
    pltpu.sync_copy(rows_v, out_hbm.at[pl.ds(base, _QPW)])


@functools.cache
def _sc_gather():
    return functools.partial(
        pl.kernel,
        out_type=jax.ShapeDtypeStruct((_Q, _D), jnp.float32),
        mesh=plsc.VectorSubcoreMesh(
            core_axis_name="c", subcore_axis_name="s",
            num_cores=_NC, num_subcores=_NS),
        scratch_types=[
            pltpu.VMEM((_QPW,), jnp.int32),
            pltpu.VMEM((_QPW, _D), jnp.float32),
            pltpu.SemaphoreType.DMA,
        ],
    )(_sc_gather_body)


def kernel(x, weight):
    B, D, H, W = x.shape
    x3 = x.reshape(B, D, H * W)
    amin2, wt = _tc_scores(x3, weight)
    rows = _sc_gather()(wt, amin2.reshape(-1))
    result = rows.reshape(B, H, W, D).transpose(0, 3, 1, 2)
    return result, amin2.reshape(B, H, W)


# speedup vs baseline: 1.0026x; 1.0026x over previous
import functools

import jax
import jax.numpy as jnp
from jax import lax
from jax.experimental import pallas as pl
from jax.experimental.pallas import tpu as pltpu
from jax.experimental.pallas import tpu_sc as plsc

_NC, _NS = 1, 16
_NW = _NC * _NS
_B = 2
_HW = 256
_Q = _B * _HW
_D = 256
_N = 512
_QPW = _Q // _NW


def _tc_scores_body(x_ref, w_ref, amin_ref, wt_ref):
    w = w_ref[...]
    e2 = jnp.sum(w * w, axis=0, keepdims=True)
    for b in range(_B):
        xb = x_ref[b]
        s = lax.dot_general(
            xb, w, (((0,), (0,)), ((), ())),
            preferred_element_type=jnp.float32,
            precision=lax.Precision.HIGHEST)
        score = e2 - 2.0 * s
        mn = jnp.min(score, axis=1, keepdims=True)
        ii = lax.broadcasted_iota(jnp.int32, score.shape, 1)
        amin_ref[b, :] = jnp.min(jnp.where(score <= mn, ii, _N), axis=1)
    wt_ref[...] = w.T


_tc_scores = pl.pallas_call(
    _tc_scores_body,
    out_shape=(
        jax.ShapeDtypeStruct((_B, _HW), jnp.int32),
        jax.ShapeDtypeStruct((_N, _D), jnp.float32),
    ),
)


def _sc_gather_body(wt_hbm, idx_hbm, out_hbm, idx_v, rows_v, sem):
    wid = lax.axis_index("s") * _NC + lax.axis_index("c")
    base = wid * _QPW
    pltpu.sync_copy(idx_hbm.at[pl.ds(base, _QPW)], idx_v)
    pltpu.async_copy(wt_hbm.at[idx_v], rows_v, sem).wait()
    pltpu.sync_copy(rows_v, out_hbm.at[pl.ds(base, _QPW)])


@functools.cache
def _sc_gather():
    return functools.partial(
        pl.kernel,
        out_type=jax.ShapeDtypeStruct((_Q, _D), jnp.float32),
        mesh=plsc.VectorSubcoreMesh(
            core_axis_name="c", subcore_axis_name="s",
            num_cores=_NC, num_subcores=_NS),
        compiler_params=pltpu.CompilerParams(skip_device_barrier=True),
        scratch_types=[
            pltpu.VMEM((_QPW,), jnp.int32),
            pltpu.VMEM((_QPW, _D), jnp.float32),
            pltpu.SemaphoreType.DMA,
        ],
    )(_sc_gather_body)


def kernel(x, weight):
    B, D, H, W = x.shape
    x3 = x.reshape(B, D, H * W)
    amin2, wt = _tc_scores(x3, weight)
    rows = _sc_gather()(wt, amin2.reshape(-1))
    result = rows.reshape(B, H, W, D).transpose(0, 3, 1, 2)
    return result, amin2.reshape(B, H, W)
